# baseline (speedup 1.0000x reference)
import functools

import jax
import jax.numpy as jnp
from jax import lax
from jax.experimental import pallas as pl
from jax.experimental.pallas import tpu as pltpu
from jax.experimental.pallas import tpu_sc as plsc

N_FIELDS = 26
VOCAB = 100000
EMB = 50
BATCH = 16384
NW = 32
PAIRS = N_FIELDS * EMB
PAIRS_PER_W = -(-PAIRS // NW)
LANES = 16
BCHUNK = 4096
N_BCHUNK = BATCH // BCHUNK


def _sc_body(x_hbm, tab_hbm, out_hbm, row_v, idx_v, out_v, osem, rsem):
  wid = lax.axis_index("s") * 2 + lax.axis_index("c")
  base_n = PAIRS // NW
  extra = PAIRS % NW
  start = wid * base_n + jnp.minimum(wid, extra)
  count = base_n + (wid < extra).astype(jnp.int32)

  def pair_body(i, prev_f):
    pair = start + i
    f = pair // EMB
    d = pair % EMB
    pltpu.async_copy(tab_hbm.at[f, d], row_v, rsem.at[0])

    @pl.when(f != prev_f)
    def _():
      pltpu.sync_copy(x_hbm.at[f], idx_v)

    pltpu.make_async_copy(tab_hbm.at[f, d], row_v, rsem.at[0]).wait()
    for cb in range(N_BCHUNK):
      p = cb % 2
      if cb >= 2:
        pltpu.make_async_copy(
            out_v.at[p],
            out_hbm.at[f, d, pl.ds((cb - 2) * BCHUNK, BCHUNK)],
            osem.at[p],
        ).wait()
      else:
        @pl.when(i > 0)
        def _():
          pltpu.make_async_copy(
              out_v.at[p],
              out_hbm.at[f, d, pl.ds(cb * BCHUNK, BCHUNK)],
              osem.at[p],
          ).wait()

      @plsc.parallel_loop(0, BCHUNK, step=LANES * 4, unroll=4)
      def _(j):
        for u in range(4):
          v16 = idx_v[pl.ds(cb * BCHUNK + j + u * LANES, LANES)]
          vals = plsc.load_gather(row_v, [v16])
          out_v[p, pl.ds(j + u * LANES, LANES)] = jnp.maximum(vals, 0.0)

      pltpu.async_copy(
          out_v.at[p], out_hbm.at[f, d, pl.ds(cb * BCHUNK, BCHUNK)], osem.at[p]
      )
    return f

  lax.fori_loop(0, count, pair_body, jnp.int32(-1))
  last = start + count - 1
  lf = last // EMB
  ld = last % EMB
  for cb in (N_BCHUNK - 2, N_BCHUNK - 1):
    pltpu.make_async_copy(
        out_v.at[cb % 2],
        out_hbm.at[lf, ld, pl.ds(cb * BCHUNK, BCHUNK)],
        osem.at[cb % 2],
    ).wait()


@jax.jit
def kernel(x, tables):
  xi = x.astype(jnp.int32)
  tab_t = jnp.transpose(tables, (0, 2, 1))
  mesh = plsc.VectorSubcoreMesh(core_axis_name="c", subcore_axis_name="s")
  out_t = pl.kernel(
      _sc_body,
      out_type=jax.ShapeDtypeStruct((N_FIELDS, EMB, BATCH), jnp.float32),
      mesh=mesh,
      scratch_types=[
          pltpu.VMEM((VOCAB,), jnp.float32),
          pltpu.VMEM((BATCH,), jnp.int32),
          pltpu.VMEM((2, BCHUNK), jnp.float32),
          pltpu.SemaphoreType.DMA((2,)),
          pltpu.SemaphoreType.DMA((2,)),
      ],
      compiler_params=pltpu.CompilerParams(
          use_tc_tiling_on_sc=True, needs_layout_passes=False
      ),
  )(xi, tab_t)
  return jnp.transpose(out_t, (0, 2, 1))


# speedup vs baseline: 1.0017x; 1.0017x over previous
import jax
import jax.numpy as jnp
from jax import lax
from jax.experimental import pallas as pl
from jax.experimental.pallas import tpu as pltpu
from jax.experimental.pallas import tpu_sc as plsc

N_FIELDS = 26
VOCAB = 100000
EMB = 50
BATCH = 16384
NW = 32
PAIRS = N_FIELDS * EMB
LANES = 16
BCHUNK = 4096
N_BCHUNK = BATCH // BCHUNK


def _sc_body(x_hbm, tab_hbm, out_hbm, row_v, idx_v, out_v, osem, rsem):
  wid = lax.axis_index("s") * 2 + lax.axis_index("c")
  base_n = PAIRS // NW
  extra = PAIRS % NW
  start = wid * base_n + jnp.minimum(wid, extra)
  count = base_n + (wid < extra).astype(jnp.int32)

  def pair_body(i, prev_f):
    pair = start + i
    f = pair // EMB
    d = pair % EMB
    pltpu.async_copy(tab_hbm.at[f, d], row_v, rsem.at[0])

    @pl.when(f != prev_f)
    def _():
      pltpu.sync_copy(x_hbm.at[f], idx_v)

    pltpu.make_async_copy(tab_hbm.at[f, d], row_v, rsem.at[0]).wait()
    for cb in range(N_BCHUNK):
      p = cb % 2
      if cb >= 2:
        pltpu.make_async_copy(
            out_v.at[p],
            out_hbm.at[f, d, pl.ds((cb - 2) * BCHUNK, BCHUNK)],
            osem.at[p],
        ).wait()
      else:
        @pl.when(i > 0)
        def _():
          pltpu.make_async_copy(
              out_v.at[p],
              out_hbm.at[f, d, pl.ds(cb * BCHUNK, BCHUNK)],
              osem.at[p],
          ).wait()

      @plsc.parallel_loop(0, BCHUNK, step=LANES * 4, unroll=4)
      def _(j):
        for u in range(4):
          v16 = idx_v[pl.ds(cb * BCHUNK + j + u * LANES, LANES)]
          vals = plsc.load_gather(row_v, [v16])
          out_v[p, pl.ds(j + u * LANES, LANES)] = jnp.maximum(vals, 0.0)

      pltpu.async_copy(
          out_v.at[p], out_hbm.at[f, d, pl.ds(cb * BCHUNK, BCHUNK)], osem.at[p]
      )
    return f

  lax.fori_loop(0, count, pair_body, jnp.int32(-1))
  last = start + count - 1
  lf = last // EMB
  ld = last % EMB
  for cb in (N_BCHUNK - 2, N_BCHUNK - 1):
    pltpu.make_async_copy(
        out_v.at[cb % 2],
        out_hbm.at[lf, ld, pl.ds(cb * BCHUNK, BCHUNK)],
        osem.at[cb % 2],
    ).wait()


@jax.jit
def kernel(x, tables):
  xi = x.astype(jnp.int32)
  tab_t = jnp.transpose(tables, (0, 2, 1))
  mesh = plsc.VectorSubcoreMesh(core_axis_name="c", subcore_axis_name="s")
  out_t = pl.kernel(
      _sc_body,
      out_type=jax.ShapeDtypeStruct((N_FIELDS, EMB, BATCH), jnp.float32),
      mesh=mesh,
      scratch_types=[
          pltpu.VMEM((VOCAB,), jnp.float32),
          pltpu.VMEM((BATCH,), jnp.int32),
          pltpu.VMEM((2, BCHUNK), jnp.float32),
          pltpu.SemaphoreType.DMA((2,)),
          pltpu.SemaphoreType.DMA((2,)),
      ],
      compiler_params=pltpu.CompilerParams(
          use_tc_tiling_on_sc=True, needs_layout_passes=False
      ),
  )(xi, tab_t)
  return jnp.transpose(out_t, (0, 2, 1))
